# baseline (speedup 1.0000x reference)
import dataclasses

import jax
import jax.numpy as jnp
from jax import lax
from jax.experimental import pallas as pl
from jax.experimental.pallas import tpu as pltpu
from jax.experimental.pallas import tpu_sc as plsc

D = 128
NC = 2
NS = 16
NW = NC * NS
CH = 128
IDXB = 8
ZB = 8


def _sc_aggregate(x, src2d, dst2d, n_pad, with_cnt):
    n_chunks = src2d.shape[0]
    cpw = n_chunks // NW
    nblk = cpw // IDXB
    rpw = n_pad // NS

    mesh = plsc.VectorSubcoreMesh(core_axis_name="c", subcore_axis_name="s")
    out_type = [jax.ShapeDtypeStruct((NC, n_pad, D), jnp.float32)]
    scratch = [
        pltpu.VMEM((IDXB, CH), jnp.int32),
        pltpu.VMEM((IDXB, CH), jnp.int32),
        pltpu.VMEM((IDXB, CH), jnp.int32),
        pltpu.VMEM((IDXB, CH), jnp.int32),
        pltpu.VMEM((CH, D), jnp.float32),
        pltpu.VMEM((CH, D), jnp.float32),
        pltpu.VMEM((ZB, D), jnp.float32),
        pltpu.VMEM_SHARED((n_pad, D), jnp.float32),
        pltpu.SemaphoreType.DMA,
        pltpu.SemaphoreType.DMA,
        pltpu.SemaphoreType.DMA,
        pltpu.SemaphoreType.DMA,
        pltpu.SemaphoreType.DMA,
        pltpu.SemaphoreType.DMA,
    ]
    def body(x_hbm, src_hbm, dst_hbm, *rest):
        (out_hbm, srcv0, srcv1, dstv0, dstv1, bufa, bufb, zbuf,
         acc, sema, semb, isem, zsem, ssema, ssemb) = rest
        cid = lax.axis_index("c")
        sid = lax.axis_index("s")
        wid = sid * NC + cid
        rbase = sid * rpw
        ebase = wid * cpw
        sv = [srcv0, srcv1]
        dv = [dstv0, dstv1]

        @pl.loop(0, ZB)
        def _(i):
            @pl.loop(0, D // 16)
            def _(j):
                zbuf[i, pl.ds(j * 16, 16)] = jnp.zeros((16,), jnp.float32)

        nz = rpw // ZB

        @pl.loop(0, nz)
        def _(r):
            pltpu.make_async_copy(
                zbuf, acc.at[pl.ds(rbase + r * ZB, ZB)], zsem).start()

        @pl.loop(0, nz)
        def _(r):
            pltpu.make_async_copy(
                zbuf, acc.at[pl.ds(rbase + r * ZB, ZB)], zsem).wait()

        def idx_copy(blkno, pp):
            row = ebase + blkno * IDXB
            return (pltpu.make_async_copy(
                        src_hbm.at[pl.ds(row, IDXB)], sv[pp], isem),
                    pltpu.make_async_copy(
                        dst_hbm.at[pl.ds(row, IDXB)], dv[pp], isem))

        cs, cd = idx_copy(0, 0)
        cs.start()
        cd.start()
        cs.wait()
        cd.wait()
        if nblk > 1:
            cs, cd = idx_copy(1, 1)
            cs.start()
            cd.start()

        plsc.subcore_barrier()

        def g_copy(buf, sem, idxref, c):
            return pltpu.make_async_copy(x_hbm.at[idxref.at[c]], buf, sem)

        def s_start(buf, sem, d_, c):
            pltpu.async_copy(buf, acc.at[d_.at[c]], sem, add=True)

        def s_wait(buf, sem, d_, c):
            pltpu.make_async_copy(buf, acc.at[d_.at[c]], sem).wait()

        g_copy(bufa, sema, sv[0], 0).start()
        g_copy(bufb, semb, sv[0], 1).start()

        for b in range(nblk):
            s_, d_ = sv[b % 2], dv[b % 2]
            if b > 0 and b + 1 < nblk:
                cs, cd = idx_copy(b + 1, (b + 1) % 2)
                cs.start()
                cd.start()

            @pl.loop(0, IDXB, step=2)
            def _(c, s_=s_, d_=d_):
                g_copy(bufa, sema, s_, c).wait()
                s_start(bufa, ssema, d_, c)
                g_copy(bufb, semb, s_, c + 1).wait()
                s_start(bufb, ssemb, d_, c + 1)

                @pl.when(c + 2 < IDXB)
                def _():
                    s_wait(bufa, ssema, d_, c)
                    g_copy(bufa, sema, s_, c + 2).start()

                @pl.when(c + 3 < IDXB)
                def _():
                    s_wait(bufb, ssemb, d_, c + 1)
                    g_copy(bufb, semb, s_, c + 3).start()

            if b + 1 < nblk:
                nxt = (b + 1) % 2
                cs, cd = idx_copy(b + 1, nxt)
                cs.wait()
                cd.wait()
                s_wait(bufa, ssema, d_, IDXB - 2)
                g_copy(bufa, sema, sv[nxt], 0).start()
                s_wait(bufb, ssemb, d_, IDXB - 1)
                g_copy(bufb, semb, sv[nxt], 1).start()

        s_wait(bufa, ssema, dv[(nblk - 1) % 2], IDXB - 2)
        s_wait(bufb, ssemb, dv[(nblk - 1) % 2], IDXB - 1)

        plsc.subcore_barrier()

        pltpu.sync_copy(acc.at[pl.ds(rbase, rpw)],
                        out_hbm.at[cid, pl.ds(rbase, rpw)])

    fn = pl.kernel(body, out_type=out_type, mesh=mesh, scratch_types=scratch)
    return fn(x, src2d, dst2d)


def _sc_counts(dst2d, n_pad):
    n_chunks = dst2d.shape[0]
    cpw = n_chunks // NW

    mesh = plsc.VectorSubcoreMesh(core_axis_name="c", subcore_axis_name="s")
    scratch = [
        pltpu.VMEM((cpw, CH), jnp.int32),
        pltpu.VMEM((1, n_pad), jnp.float32),
    ]

    def body(dst_hbm, cnt_hbm, dstv, hist):
        cid = lax.axis_index("c")
        sid = lax.axis_index("s")
        wid = sid * NC + cid
        ebase = wid * cpw

        @pl.loop(0, n_pad // 16)
        def _(i):
            hist[0, pl.ds(i * 16, 16)] = jnp.zeros((16,), jnp.float32)

        pltpu.sync_copy(dst_hbm.at[pl.ds(ebase, cpw)], dstv)

        ones16 = jnp.ones((16,), jnp.float32)
        zero16 = jnp.zeros((16,), jnp.int32)

        @pl.loop(0, cpw)
        def _(c):
            @pl.loop(0, CH // 16)
            def _(j):
                plsc.addupdate_scatter(
                    hist, [zero16, dstv[c, pl.ds(j * 16, 16)]], ones16)

        pltpu.sync_copy(hist, cnt_hbm.at[cid, pl.ds(sid, 1)])

    cp = pltpu.CompilerParams()
    if "needs_layout_passes" in pltpu.CompilerParams.__dataclass_fields__:
        cp = dataclasses.replace(cp, needs_layout_passes=False)
    fn = pl.kernel(body,
                   out_type=jax.ShapeDtypeStruct((NC, NS, n_pad), jnp.float32),
                   mesh=mesh, scratch_types=scratch, compiler_params=cp)
    return fn(dst2d)


def _tc_combine(p, cnt, h, wl_t, wr_t, bias, n, n_pad, blk):
    def body(p_ref, c_ref, h_ref, wl_ref, wr_ref, b_ref, o_ref):
        i = pl.program_id(0)
        c = lax.dot_general(c_ref[:, pl.ds(i * blk, blk)],
                            jnp.ones((NW, D), jnp.float32),
                            (((0,), (0,)), ((), ())),
                            preferred_element_type=jnp.float32)
        agg = (p_ref[0] + p_ref[1]) / jnp.maximum(c, 1.0)
        o_ref[...] = jax.nn.relu(
            jnp.dot(agg, wl_ref[...], preferred_element_type=jnp.float32)
            + b_ref[0]
            + jnp.dot(h_ref[...], wr_ref[...],
                      preferred_element_type=jnp.float32))

    return pl.pallas_call(
        body,
        grid=(-(-n // blk),),
        in_specs=[
            pl.BlockSpec((NC, blk, D), lambda i: (0, i, 0)),
            pl.BlockSpec((NW, n_pad), lambda i: (0, 0)),
            pl.BlockSpec((blk, D), lambda i: (i, 0)),
            pl.BlockSpec((D, D), lambda i: (0, 0)),
            pl.BlockSpec((D, D), lambda i: (0, 0)),
            pl.BlockSpec((1, D), lambda i: (0, 0)),
        ],
        out_specs=pl.BlockSpec((blk, D), lambda i: (i, 0)),
        out_shape=jax.ShapeDtypeStruct((n, D), jnp.float32),
    )(p, cnt, h, wl_t, wr_t, bias)


def kernel(x, edge_index, W1l, b1l, W1r, W2l, b2l, W2r):
    n, d = x.shape
    e = edge_index.shape[1]
    assert d == D

    n_pad = -(-(n + 1) // (NS * ZB * 2)) * (NS * ZB * 2)
    epw = -(-e // (NW * IDXB * CH)) * (IDXB * CH)
    e_pad = NW * epw

    src = edge_index[0]
    dst = edge_index[1]
    pad = e_pad - e
    src_p = jnp.concatenate([src, jnp.zeros((pad,), jnp.int32)])
    dst_p = jnp.concatenate([dst, jnp.full((pad,), n, jnp.int32)])
    src2d = src_p.reshape(-1, CH)
    dst2d = dst_p.reshape(-1, CH)

    blk = 2048
    b1 = b1l.reshape(1, D)
    b2 = b2l.reshape(1, D)

    cnt = _sc_counts(dst2d, n_pad).reshape(NW, n_pad)
    (p1,) = _sc_aggregate(x, src2d, dst2d, n_pad, with_cnt=False)
    h1 = _tc_combine(p1, cnt, x, W1l.T, W1r.T, b1, n, n_pad, blk)
    (p2,) = _sc_aggregate(h1, src2d, dst2d, n_pad, with_cnt=False)
    return _tc_combine(p2, cnt, h1, W2l.T, W2r.T, b2, n, n_pad, blk)


# speedup vs baseline: 1.0032x; 1.0032x over previous
import dataclasses

import jax
import jax.numpy as jnp
from jax import lax
from jax.experimental import pallas as pl
from jax.experimental.pallas import tpu as pltpu
from jax.experimental.pallas import tpu_sc as plsc

D = 128
NC = 2
NS = 16
NW = NC * NS
CH = 128
IDXB = 8
ZB = 8


def _sc_aggregate(x, src2d, dst2d, n_pad, with_cnt):
    n_chunks = src2d.shape[0]
    cpw = n_chunks // NW
    nblk = cpw // IDXB
    rpw = n_pad // NS

    mesh = plsc.VectorSubcoreMesh(core_axis_name="c", subcore_axis_name="s")
    out_type = [jax.ShapeDtypeStruct((NC, n_pad, D), jnp.float32)]
    scratch = [
        pltpu.VMEM((IDXB, CH), jnp.int32),
        pltpu.VMEM((IDXB, CH), jnp.int32),
        pltpu.VMEM((IDXB, CH), jnp.int32),
        pltpu.VMEM((IDXB, CH), jnp.int32),
        pltpu.VMEM((CH, D), jnp.float32),
        pltpu.VMEM((CH, D), jnp.float32),
        pltpu.VMEM((ZB, D), jnp.float32),
        pltpu.VMEM_SHARED((n_pad, D), jnp.float32),
        pltpu.SemaphoreType.DMA,
        pltpu.SemaphoreType.DMA,
        pltpu.SemaphoreType.DMA,
        pltpu.SemaphoreType.DMA,
        pltpu.SemaphoreType.DMA,
        pltpu.SemaphoreType.DMA,
    ]
    def body(x_hbm, src_hbm, dst_hbm, *rest):
        (out_hbm, srcv0, srcv1, dstv0, dstv1, bufa, bufb, zbuf,
         acc, sema, semb, isem, zsem, ssema, ssemb) = rest
        cid = lax.axis_index("c")
        sid = lax.axis_index("s")
        wid = sid * NC + cid
        rbase = sid * rpw
        ebase = wid * cpw
        sv = [srcv0, srcv1]
        dv = [dstv0, dstv1]

        @pl.loop(0, ZB)
        def _(i):
            @pl.loop(0, D // 16)
            def _(j):
                zbuf[i, pl.ds(j * 16, 16)] = jnp.zeros((16,), jnp.float32)

        nz = rpw // ZB

        @pl.loop(0, nz)
        def _(r):
            pltpu.make_async_copy(
                zbuf, acc.at[pl.ds(rbase + r * ZB, ZB)], zsem).start()

        @pl.loop(0, nz)
        def _(r):
            pltpu.make_async_copy(
                zbuf, acc.at[pl.ds(rbase + r * ZB, ZB)], zsem).wait()

        def idx_copy(blkno, pp):
            row = ebase + blkno * IDXB
            return (pltpu.make_async_copy(
                        src_hbm.at[pl.ds(row, IDXB)], sv[pp], isem),
                    pltpu.make_async_copy(
                        dst_hbm.at[pl.ds(row, IDXB)], dv[pp], isem))

        cs, cd = idx_copy(0, 0)
        cs.start()
        cd.start()
        cs.wait()
        cd.wait()
        if nblk > 1:
            cs, cd = idx_copy(1, 1)
            cs.start()
            cd.start()

        plsc.subcore_barrier()

        def g_copy(buf, sem, idxref, c):
            return pltpu.make_async_copy(x_hbm.at[idxref.at[c]], buf, sem)

        def s_start(buf, sem, d_, c):
            pltpu.async_copy(buf, acc.at[d_.at[c]], sem, add=True)

        def s_wait(buf, sem, d_, c):
            pltpu.make_async_copy(buf, acc.at[d_.at[c]], sem).wait()

        g_copy(bufa, sema, sv[0], 0).start()
        g_copy(bufb, semb, sv[0], 1).start()

        for b in range(nblk):
            s_, d_ = sv[b % 2], dv[b % 2]
            if b > 0 and b + 1 < nblk:
                cs, cd = idx_copy(b + 1, (b + 1) % 2)
                cs.start()
                cd.start()

            @pl.loop(0, IDXB, step=2)
            def _(c, s_=s_, d_=d_):
                g_copy(bufa, sema, s_, c).wait()
                s_start(bufa, ssema, d_, c)
                g_copy(bufb, semb, s_, c + 1).wait()
                s_start(bufb, ssemb, d_, c + 1)

                @pl.when(c + 2 < IDXB)
                def _():
                    s_wait(bufa, ssema, d_, c)
                    g_copy(bufa, sema, s_, c + 2).start()

                @pl.when(c + 3 < IDXB)
                def _():
                    s_wait(bufb, ssemb, d_, c + 1)
                    g_copy(bufb, semb, s_, c + 3).start()

            if b + 1 < nblk:
                nxt = (b + 1) % 2
                cs, cd = idx_copy(b + 1, nxt)
                cs.wait()
                cd.wait()
                s_wait(bufa, ssema, d_, IDXB - 2)
                g_copy(bufa, sema, sv[nxt], 0).start()
                s_wait(bufb, ssemb, d_, IDXB - 1)
                g_copy(bufb, semb, sv[nxt], 1).start()

        s_wait(bufa, ssema, dv[(nblk - 1) % 2], IDXB - 2)
        s_wait(bufb, ssemb, dv[(nblk - 1) % 2], IDXB - 1)

        plsc.subcore_barrier()

        pltpu.sync_copy(acc.at[pl.ds(rbase, rpw)],
                        out_hbm.at[cid, pl.ds(rbase, rpw)])

    fn = pl.kernel(body, out_type=out_type, mesh=mesh, scratch_types=scratch)
    return fn(x, src2d, dst2d)


def _sc_counts(dst2d, n_pad):
    n_chunks = dst2d.shape[0]
    cpw = n_chunks // NW

    mesh = plsc.VectorSubcoreMesh(core_axis_name="c", subcore_axis_name="s")
    scratch = [
        pltpu.VMEM((cpw, CH), jnp.int32),
        pltpu.VMEM((1, n_pad), jnp.float32),
    ]

    def body(dst_hbm, cnt_hbm, dstv, hist):
        cid = lax.axis_index("c")
        sid = lax.axis_index("s")
        wid = sid * NC + cid
        ebase = wid * cpw

        @pl.loop(0, n_pad // 16)
        def _(i):
            hist[0, pl.ds(i * 16, 16)] = jnp.zeros((16,), jnp.float32)

        pltpu.sync_copy(dst_hbm.at[pl.ds(ebase, cpw)], dstv)

        ones16 = jnp.ones((16,), jnp.float32)
        zero16 = jnp.zeros((16,), jnp.int32)

        @pl.loop(0, cpw)
        def _(c):
            @pl.loop(0, CH // 16)
            def _(j):
                plsc.addupdate_scatter(
                    hist, [zero16, dstv[c, pl.ds(j * 16, 16)]], ones16)

        pltpu.sync_copy(hist, cnt_hbm.at[cid, pl.ds(sid, 1)])

    cp = pltpu.CompilerParams()
    if "needs_layout_passes" in pltpu.CompilerParams.__dataclass_fields__:
        cp = dataclasses.replace(cp, needs_layout_passes=False)
    fn = pl.kernel(body,
                   out_type=jax.ShapeDtypeStruct((NC, NS, n_pad), jnp.float32),
                   mesh=mesh, scratch_types=scratch, compiler_params=cp)
    return fn(dst2d)


def _tc_combine(p, cnt, h, wl_t, wr_t, bias, n, n_pad, blk):
    def body(p_ref, c_ref, h_ref, wl_ref, wr_ref, b_ref, o_ref):
        i = pl.program_id(0)
        c = lax.dot_general(c_ref[:, pl.ds(i * blk, blk)],
                            jnp.ones((NW, D), jnp.float32),
                            (((0,), (0,)), ((), ())),
                            preferred_element_type=jnp.float32)
        agg = (p_ref[0] + p_ref[1]) / jnp.maximum(c, 1.0)
        o_ref[...] = jax.nn.relu(
            jnp.dot(agg, wl_ref[...], preferred_element_type=jnp.float32)
            + b_ref[0]
            + jnp.dot(h_ref[...], wr_ref[...],
                      preferred_element_type=jnp.float32))

    return pl.pallas_call(
        body,
        grid=(-(-n // blk),),
        in_specs=[
            pl.BlockSpec((NC, blk, D), lambda i: (0, i, 0)),
            pl.BlockSpec((NW, n_pad), lambda i: (0, 0)),
            pl.BlockSpec((blk, D), lambda i: (i, 0)),
            pl.BlockSpec((D, D), lambda i: (0, 0)),
            pl.BlockSpec((D, D), lambda i: (0, 0)),
            pl.BlockSpec((1, D), lambda i: (0, 0)),
        ],
        out_specs=pl.BlockSpec((blk, D), lambda i: (i, 0)),
        out_shape=jax.ShapeDtypeStruct((n, D), jnp.float32),
    )(p, cnt, h, wl_t, wr_t, bias)


def kernel(x, edge_index, W1l, b1l, W1r, W2l, b2l, W2r):
    n, d = x.shape
    e = edge_index.shape[1]
    assert d == D

    n_pad = -(-(n + 1) // (NS * ZB * 2)) * (NS * ZB * 2)
    epw = -(-e // (NW * IDXB * CH)) * (IDXB * CH)
    e_pad = NW * epw

    src = edge_index[0]
    dst = edge_index[1]
    pad = e_pad - e
    src_p = jnp.concatenate([src, jnp.zeros((pad,), jnp.int32)])
    dst_p = jnp.concatenate(
        [dst, n + jnp.arange(pad, dtype=jnp.int32) % (n_pad - n)])
    src2d = src_p.reshape(-1, CH)
    dst2d = dst_p.reshape(-1, CH)

    blk = 2048
    b1 = b1l.reshape(1, D)
    b2 = b2l.reshape(1, D)

    cnt = _sc_counts(dst2d, n_pad).reshape(NW, n_pad)
    (p1,) = _sc_aggregate(x, src2d, dst2d, n_pad, with_cnt=False)
    h1 = _tc_combine(p1, cnt, x, W1l.T, W1r.T, b1, n, n_pad, blk)
    (p2,) = _sc_aggregate(h1, src2d, dst2d, n_pad, with_cnt=False)
    return _tc_combine(p2, cnt, h1, W2l.T, W2r.T, b2, n, n_pad, blk)


# speedup vs baseline: 3.1917x; 3.1816x over previous
import dataclasses

import jax
import jax.numpy as jnp
from jax import lax
from jax.experimental import pallas as pl
from jax.experimental.pallas import tpu as pltpu
from jax.experimental.pallas import tpu_sc as plsc

D = 128
NC = 2
NS = 16
NW = NC * NS
CH = 128
IDXB = 8
ZB = 8


def _sc_aggregate(x, src2d, dst2d, n_pad, with_cnt):
    n_chunks = src2d.shape[0]
    cpw = n_chunks // NW
    nblk = cpw // IDXB
    rpw = n_pad // NS

    mesh = plsc.VectorSubcoreMesh(core_axis_name="c", subcore_axis_name="s")
    out_type = [jax.ShapeDtypeStruct((NC, n_pad, D), jnp.float32)]
    scratch = [
        pltpu.VMEM((IDXB, CH), jnp.int32),
        pltpu.VMEM((IDXB, CH), jnp.int32),
        pltpu.VMEM((IDXB, CH), jnp.int32),
        pltpu.VMEM((IDXB, CH), jnp.int32),
        pltpu.VMEM((CH, D), jnp.float32),
        pltpu.VMEM((CH, D), jnp.float32),
        pltpu.VMEM((ZB, D), jnp.float32),
        pltpu.VMEM_SHARED((n_pad, D), jnp.float32),
        pltpu.SemaphoreType.DMA,
        pltpu.SemaphoreType.DMA,
        pltpu.SemaphoreType.DMA,
        pltpu.SemaphoreType.DMA,
        pltpu.SemaphoreType.DMA,
        pltpu.SemaphoreType.DMA,
    ]
    def body(x_hbm, src_hbm, dst_hbm, *rest):
        (out_hbm, srcv0, srcv1, dstv0, dstv1, bufa, bufb, zbuf,
         acc, sema, semb, isem, zsem, ssema, ssemb) = rest
        cid = lax.axis_index("c")
        sid = lax.axis_index("s")
        wid = sid * NC + cid
        rbase = sid * rpw
        ebase = wid * cpw
        sv = [srcv0, srcv1]
        dv = [dstv0, dstv1]

        @pl.loop(0, ZB)
        def _(i):
            @pl.loop(0, D // 16)
            def _(j):
                zbuf[i, pl.ds(j * 16, 16)] = jnp.zeros((16,), jnp.float32)

        nz = rpw // ZB

        @pl.loop(0, nz)
        def _(r):
            pltpu.make_async_copy(
                zbuf, acc.at[pl.ds(rbase + r * ZB, ZB)], zsem).start()

        @pl.loop(0, nz)
        def _(r):
            pltpu.make_async_copy(
                zbuf, acc.at[pl.ds(rbase + r * ZB, ZB)], zsem).wait()

        def idx_copy(blkno, pp):
            row = ebase + blkno * IDXB
            return (pltpu.make_async_copy(
                        src_hbm.at[pl.ds(row, IDXB)], sv[pp], isem),
                    pltpu.make_async_copy(
                        dst_hbm.at[pl.ds(row, IDXB)], dv[pp], isem))

        cs, cd = idx_copy(0, 0)
        cs.start()
        cd.start()
        cs.wait()
        cd.wait()
        if nblk > 1:
            cs, cd = idx_copy(1, 1)
            cs.start()
            cd.start()

        plsc.subcore_barrier()

        def g_copy(buf, sem, idxref, c):
            return pltpu.make_async_copy(x_hbm.at[idxref.at[c]], buf, sem)

        def s_start(buf, sem, d_, c):
            pltpu.async_copy(buf, acc.at[d_.at[c]], sem, add=True)

        def s_wait(buf, sem, d_, c):
            pltpu.make_async_copy(buf, acc.at[d_.at[c]], sem).wait()

        g_copy(bufa, sema, sv[0], 0).start()
        g_copy(bufb, semb, sv[0], 1).start()

        for b in range(nblk):
            s_, d_ = sv[b % 2], dv[b % 2]
            if b > 0 and b + 1 < nblk:
                cs, cd = idx_copy(b + 1, (b + 1) % 2)
                cs.start()
                cd.start()

            @pl.loop(0, IDXB, step=2)
            def _(c, s_=s_, d_=d_):
                g_copy(bufa, sema, s_, c).wait()
                s_start(bufa, ssema, d_, c)
                g_copy(bufb, semb, s_, c + 1).wait()
                s_start(bufb, ssemb, d_, c + 1)

                @pl.when(c + 2 < IDXB)
                def _():
                    s_wait(bufa, ssema, d_, c)
                    g_copy(bufa, sema, s_, c + 2).start()

                @pl.when(c + 3 < IDXB)
                def _():
                    s_wait(bufb, ssemb, d_, c + 1)
                    g_copy(bufb, semb, s_, c + 3).start()

            if b + 1 < nblk:
                nxt = (b + 1) % 2
                cs, cd = idx_copy(b + 1, nxt)
                cs.wait()
                cd.wait()
                s_wait(bufa, ssema, d_, IDXB - 2)
                g_copy(bufa, sema, sv[nxt], 0).start()
                s_wait(bufb, ssemb, d_, IDXB - 1)
                g_copy(bufb, semb, sv[nxt], 1).start()

        s_wait(bufa, ssema, dv[(nblk - 1) % 2], IDXB - 2)
        s_wait(bufb, ssemb, dv[(nblk - 1) % 2], IDXB - 1)

        plsc.subcore_barrier()

        pltpu.sync_copy(acc.at[pl.ds(rbase, rpw)],
                        out_hbm.at[cid, pl.ds(rbase, rpw)])

    fn = pl.kernel(body, out_type=out_type, mesh=mesh, scratch_types=scratch)
    return fn(x, src2d, dst2d)


def _sc_counts(dst2d, n_pad):
    n_chunks = dst2d.shape[0]
    cpw = n_chunks // NW

    mesh = plsc.VectorSubcoreMesh(core_axis_name="c", subcore_axis_name="s")
    scratch = [
        pltpu.VMEM((cpw, CH), jnp.int32),
        pltpu.VMEM((1, n_pad), jnp.float32),
    ]

    def body(dst_hbm, cnt_hbm, dstv, hist):
        cid = lax.axis_index("c")
        sid = lax.axis_index("s")
        wid = sid * NC + cid
        ebase = wid * cpw

        @pl.loop(0, n_pad // 16)
        def _(i):
            hist[0, pl.ds(i * 16, 16)] = jnp.zeros((16,), jnp.float32)

        pltpu.sync_copy(dst_hbm.at[pl.ds(ebase, cpw)], dstv)

        ones16 = jnp.ones((16,), jnp.float32)
        zero16 = jnp.zeros((16,), jnp.int32)

        @pl.loop(0, cpw)
        def _(c):
            @pl.loop(0, CH // 16)
            def _(j):
                plsc.addupdate_scatter(
                    hist, [zero16, dstv[c, pl.ds(j * 16, 16)]], ones16)

        pltpu.sync_copy(hist, cnt_hbm.at[cid, pl.ds(sid, 1)])

    cp = pltpu.CompilerParams()
    if "needs_layout_passes" in pltpu.CompilerParams.__dataclass_fields__:
        cp = dataclasses.replace(cp, needs_layout_passes=False)
    fn = pl.kernel(body,
                   out_type=jax.ShapeDtypeStruct((NC, NS, n_pad), jnp.float32),
                   mesh=mesh, scratch_types=scratch, compiler_params=cp)
    return fn(dst2d)


def _tc_combine(p, cnt, h, wl_t, wr_t, bias, n, n_pad, blk):
    def body(p_ref, c_ref, h_ref, wl_ref, wr_ref, b_ref, o_ref):
        i = pl.program_id(0)
        c = lax.dot_general(c_ref[:, pl.ds(i * blk, blk)],
                            jnp.ones((NW, D), jnp.float32),
                            (((0,), (0,)), ((), ())),
                            preferred_element_type=jnp.float32)
        agg = (p_ref[0] + p_ref[1]) / jnp.maximum(c, 1.0)
        o_ref[...] = jax.nn.relu(
            jnp.dot(agg, wl_ref[...], preferred_element_type=jnp.float32)
            + b_ref[0]
            + jnp.dot(h_ref[...], wr_ref[...],
                      preferred_element_type=jnp.float32))

    return pl.pallas_call(
        body,
        grid=(-(-n // blk),),
        in_specs=[
            pl.BlockSpec((NC, blk, D), lambda i: (0, i, 0)),
            pl.BlockSpec((NW, n_pad), lambda i: (0, 0)),
            pl.BlockSpec((blk, D), lambda i: (i, 0)),
            pl.BlockSpec((D, D), lambda i: (0, 0)),
            pl.BlockSpec((D, D), lambda i: (0, 0)),
            pl.BlockSpec((1, D), lambda i: (0, 0)),
        ],
        out_specs=pl.BlockSpec((blk, D), lambda i: (i, 0)),
        out_shape=jax.ShapeDtypeStruct((n, D), jnp.float32),
    )(p, cnt, h, wl_t, wr_t, bias)


def kernel(x, edge_index, W1l, b1l, W1r, W2l, b2l, W2r):
    n, d = x.shape
    e = edge_index.shape[1]
    assert d == D

    n_pad = -(-(n + 1) // (NS * ZB * 2)) * (NS * ZB * 2)
    epw = -(-e // (NW * IDXB * CH)) * (IDXB * CH)
    e_pad = NW * epw

    src = edge_index[0]
    dst = edge_index[1]
    pad = e_pad - e
    ramp = jnp.arange(pad, dtype=jnp.int32)
    src_p = jnp.concatenate([src, ramp % n])
    dst_p = jnp.concatenate([dst, n + ramp % (n_pad - n)])
    src2d = src_p.reshape(-1, CH)
    dst2d = dst_p.reshape(-1, CH)

    blk = 2048
    b1 = b1l.reshape(1, D)
    b2 = b2l.reshape(1, D)

    cnt = _sc_counts(dst2d, n_pad).reshape(NW, n_pad)
    (p1,) = _sc_aggregate(x, src2d, dst2d, n_pad, with_cnt=False)
    h1 = _tc_combine(p1, cnt, x, W1l.T, W1r.T, b1, n, n_pad, blk)
    (p2,) = _sc_aggregate(h1, src2d, dst2d, n_pad, with_cnt=False)
    return _tc_combine(p2, cnt, h1, W2l.T, W2r.T, b2, n, n_pad, blk)
